# Initial kernel scaffold; baseline (speedup 1.0000x reference)
import jax
import jax.numpy as jnp
from jax.experimental import pallas as pl


def kernel(x):
    raise NotImplementedError("write your pallas kernel here")


# speedup vs baseline: 2.2295x; 2.2295x over previous
import dataclasses
import functools

import jax
import jax.numpy as jnp
from jax import lax
from jax.experimental import pallas as pl
from jax.experimental.pallas import tpu as pltpu
from jax.experimental.pallas import tpu_sc as plsc

R, C = 128, 32768
K = 64
L = 16
NWORKERS = 32
ROWS_PER = R // NWORKERS

H1_BITS, H2_BITS, H3_BITS = 12, 12, 7
H1, H2, H3 = 1 << H1_BITS, 1 << H2_BITS, 1 << H3_BITS


def _zero_hist(h, nbins):
    zeros = jnp.zeros((L,), jnp.int32)

    @pl.loop(0, nbins, step=L)
    def _(i):
        h[pl.ds(i, L)] = zeros


def _scan_hist(h, nbins, rank):
    nv = nbins // L

    def cond(st):
        _, cum = st
        return cum < rank

    def body(st):
        v, cum = st
        s = jnp.sum(h[pl.ds(v * L, L)])
        return (v - 1, cum + s)

    v_end, cum = lax.while_loop(cond, body,
                                (jnp.int32(nv - 1), jnp.int32(0)))
    vstar = v_end + 1
    hv = h[pl.ds(vstar * L, L)]
    s = jnp.sum(hv)
    prev = cum - s
    rev = lax.rev(hv, (0,))
    csum = plsc.cumsum(rev)
    need = rank - prev
    lane = plsc.all_reduce_ffs(csum >= need)
    li = lax.iota(jnp.int32, L)
    c_at = jnp.sum(jnp.where(li == lane, csum, 0))
    h_at = jnp.sum(jnp.where(li == lane, rev, 0))
    bin_index = vstar * L + (L - 1 - lane)
    above = prev + c_at - h_at
    return bin_index, rank - above


def _do_row(row_v, h1, h2, h3):
    ones = jnp.ones((L,), jnp.int32)

    _zero_hist(h1, H1)
    _zero_hist(h2, H2)
    _zero_hist(h3, H3)

    @pl.loop(0, C, step=L)
    def _(i):
        r = jnp.maximum(row_v[pl.ds(i, L)], 0.0)
        k = plsc.bitcast(r, jnp.int32)
        plsc.addupdate_scatter(
            h1, [jnp.right_shift(k, H2_BITS + H3_BITS)], ones)

    b1, rank2 = _scan_hist(h1, H1, jnp.int32(K))

    @pl.loop(0, C, step=L)
    def _(i):
        r = jnp.maximum(row_v[pl.ds(i, L)], 0.0)
        k = plsc.bitcast(r, jnp.int32)
        m = jnp.right_shift(k, H2_BITS + H3_BITS) == b1
        plsc.addupdate_scatter(
            h2, [jnp.bitwise_and(jnp.right_shift(k, H3_BITS), H2 - 1)],
            ones, mask=m)

    b2, rank3 = _scan_hist(h2, H2, rank2)
    prefix = jnp.bitwise_or(lax.shift_left(b1, H2_BITS), b2)

    @pl.loop(0, C, step=L)
    def _(i):
        r = jnp.maximum(row_v[pl.ds(i, L)], 0.0)
        k = plsc.bitcast(r, jnp.int32)
        m = jnp.right_shift(k, H3_BITS) == prefix
        plsc.addupdate_scatter(
            h3, [jnp.bitwise_and(k, H3 - 1)], ones, mask=m)

    b3, rank_eq = _scan_hist(h3, H3, rank3)
    tkey = jnp.bitwise_or(lax.shift_left(prefix, H3_BITS), b3)

    def out_body(iv, eqcount):
        i = iv * L
        r = jnp.maximum(row_v[pl.ds(i, L)], 0.0)
        k = plsc.bitcast(r, jnp.int32)
        meq = k == tkey
        meqi = meq.astype(jnp.int32)
        pc = plsc.cumsum(meqi)
        occ = eqcount + pc - meqi
        keep = jnp.logical_or(k > tkey, jnp.logical_and(meq, occ < rank_eq))
        row_v[pl.ds(i, L)] = jnp.where(keep, r, 0.0)
        return eqcount + jnp.sum(meqi)

    lax.fori_loop(0, C // L, out_body, jnp.int32(0))


def kernel(x):
    mesh = plsc.VectorSubcoreMesh(core_axis_name="c", subcore_axis_name="s")
    cp = pltpu.CompilerParams()
    if "needs_layout_passes" in pltpu.CompilerParams.__dataclass_fields__:
        cp = dataclasses.replace(cp, needs_layout_passes=False)

    @functools.partial(
        pl.kernel,
        out_type=jax.ShapeDtypeStruct((R, C), jnp.float32),
        mesh=mesh,
        compiler_params=cp,
        scratch_types=[
            pltpu.VMEM((C,), jnp.float32),
            pltpu.VMEM((H1,), jnp.int32),
            pltpu.VMEM((H2,), jnp.int32),
            pltpu.VMEM((H3,), jnp.int32),
        ],
    )
    def k(x_hbm, o_hbm, row_v, h1, h2, h3):
        wid = lax.axis_index("s") * 2 + lax.axis_index("c")

        @pl.loop(0, ROWS_PER)
        def _(j):
            row = wid * ROWS_PER + j
            pltpu.sync_copy(x_hbm.at[row], row_v)
            _do_row(row_v, h1, h2, h3)
            pltpu.sync_copy(row_v, o_hbm.at[row])

    return k(x)


# speedup vs baseline: 4.0350x; 1.8098x over previous
import dataclasses
import functools

import jax
import jax.numpy as jnp
from jax import lax
from jax.experimental import pallas as pl
from jax.experimental.pallas import tpu as pltpu
from jax.experimental.pallas import tpu_sc as plsc

R, C = 128, 32768
K = 64
L = 16
NWORKERS = 32
ROWS_PER = R // NWORKERS

H1_BITS, H2_BITS, H3_BITS = 12, 12, 7
H1, H2, H3 = 1 << H1_BITS, 1 << H2_BITS, 1 << H3_BITS


def _zero_hist(h, nbins):
    zeros = jnp.zeros((L,), jnp.int32)

    @plsc.parallel_loop(0, nbins, L, unroll=8)
    def _(i):
        h[pl.ds(i, L)] = zeros


def _scan_hist(h, nbins, rank):
    nv = nbins // L

    def cond(st):
        _, cum = st
        return cum < rank

    def body(st):
        v, cum = st
        s = jnp.sum(h[pl.ds(v * L, L)])
        return (v - 1, cum + s)

    v_end, cum = lax.while_loop(cond, body,
                                (jnp.int32(nv - 1), jnp.int32(0)))
    vstar = v_end + 1
    hv = h[pl.ds(vstar * L, L)]
    s = jnp.sum(hv)
    prev = cum - s
    rev = lax.rev(hv, (0,))
    csum = plsc.cumsum(rev)
    need = rank - prev
    lane = plsc.all_reduce_ffs(csum >= need)
    li = lax.iota(jnp.int32, L)
    c_at = jnp.sum(jnp.where(li == lane, csum, 0))
    h_at = jnp.sum(jnp.where(li == lane, rev, 0))
    bin_index = vstar * L + (L - 1 - lane)
    above = prev + c_at - h_at
    return bin_index, rank - above


def _do_row(row_v, h1, h2, h3):
    ones = jnp.ones((L,), jnp.int32)

    _zero_hist(h1, H1)
    _zero_hist(h2, H2)
    _zero_hist(h3, H3)

    @plsc.parallel_loop(0, C, L, unroll=8)
    def _(i):
        r = jnp.maximum(row_v[pl.ds(i, L)], 0.0)
        k = plsc.bitcast(r, jnp.int32)
        plsc.addupdate_scatter(
            h1, [jnp.right_shift(k, H2_BITS + H3_BITS)], ones)

    b1, rank2 = _scan_hist(h1, H1, jnp.int32(K))

    @plsc.parallel_loop(0, C, L, unroll=8)
    def _(i):
        r = jnp.maximum(row_v[pl.ds(i, L)], 0.0)
        k = plsc.bitcast(r, jnp.int32)
        m = jnp.right_shift(k, H2_BITS + H3_BITS) == b1
        plsc.addupdate_scatter(
            h2, [jnp.bitwise_and(jnp.right_shift(k, H3_BITS), H2 - 1)],
            ones, mask=m)

    b2, rank3 = _scan_hist(h2, H2, rank2)
    prefix = jnp.bitwise_or(lax.shift_left(b1, H2_BITS), b2)

    @plsc.parallel_loop(0, C, L, unroll=8)
    def _(i):
        r = jnp.maximum(row_v[pl.ds(i, L)], 0.0)
        k = plsc.bitcast(r, jnp.int32)
        m = jnp.right_shift(k, H3_BITS) == prefix
        plsc.addupdate_scatter(
            h3, [jnp.bitwise_and(k, H3 - 1)], ones, mask=m)

    b3, rank_eq = _scan_hist(h3, H3, rank3)
    tkey = jnp.bitwise_or(lax.shift_left(prefix, H3_BITS), b3)

    def out_body(iv, eqcount):
        i = iv * L
        r = jnp.maximum(row_v[pl.ds(i, L)], 0.0)
        k = plsc.bitcast(r, jnp.int32)
        meq = k == tkey
        meqi = meq.astype(jnp.int32)
        pc = plsc.cumsum(meqi)
        occ = eqcount + pc - meqi
        keep = jnp.logical_or(k > tkey, jnp.logical_and(meq, occ < rank_eq))
        row_v[pl.ds(i, L)] = jnp.where(keep, r, 0.0)
        return eqcount + jnp.sum(meqi)

    lax.fori_loop(0, C // L, out_body, jnp.int32(0))


def kernel(x):
    mesh = plsc.VectorSubcoreMesh(core_axis_name="c", subcore_axis_name="s")
    cp = pltpu.CompilerParams()
    if "needs_layout_passes" in pltpu.CompilerParams.__dataclass_fields__:
        cp = dataclasses.replace(cp, needs_layout_passes=False)

    @functools.partial(
        pl.kernel,
        out_type=jax.ShapeDtypeStruct((R, C), jnp.float32),
        mesh=mesh,
        compiler_params=cp,
        scratch_types=[
            pltpu.VMEM((C,), jnp.float32),
            pltpu.VMEM((H1,), jnp.int32),
            pltpu.VMEM((H2,), jnp.int32),
            pltpu.VMEM((H3,), jnp.int32),
        ],
    )
    def k(x_hbm, o_hbm, row_v, h1, h2, h3):
        wid = lax.axis_index("s") * 2 + lax.axis_index("c")

        @pl.loop(0, ROWS_PER)
        def _(j):
            row = wid * ROWS_PER + j
            pltpu.sync_copy(x_hbm.at[row], row_v)
            _do_row(row_v, h1, h2, h3)
            pltpu.sync_copy(row_v, o_hbm.at[row])

    return k(x)


# speedup vs baseline: 4.0590x; 1.0060x over previous
import dataclasses
import functools

import jax
import jax.numpy as jnp
from jax import lax
from jax.experimental import pallas as pl
from jax.experimental.pallas import tpu as pltpu
from jax.experimental.pallas import tpu_sc as plsc

R, C = 128, 32768
K = 64
L = 16
NWORKERS = 32
ROWS_PER = R // NWORKERS

H1_BITS, H2_BITS, H3_BITS = 12, 12, 7
H1, H2, H3 = 1 << H1_BITS, 1 << H2_BITS, 1 << H3_BITS


def _zero_hist(h, nbins):
    zeros = jnp.zeros((L,), jnp.int32)

    @plsc.parallel_loop(0, nbins, L, unroll=8)
    def _(i):
        h[pl.ds(i, L)] = zeros


def _scan_hist(h, nbins, rank):
    nv = nbins // L

    def cond(st):
        _, cum = st
        return cum < rank

    def body(st):
        v, cum = st
        s = jnp.sum(h[pl.ds(v * L, L)])
        return (v - 1, cum + s)

    v_end, cum = lax.while_loop(cond, body,
                                (jnp.int32(nv - 1), jnp.int32(0)))
    vstar = v_end + 1
    hv = h[pl.ds(vstar * L, L)]
    s = jnp.sum(hv)
    prev = cum - s
    rev = lax.rev(hv, (0,))
    csum = plsc.cumsum(rev)
    need = rank - prev
    lane = plsc.all_reduce_ffs(csum >= need)
    li = lax.iota(jnp.int32, L)
    c_at = jnp.sum(jnp.where(li == lane, csum, 0))
    h_at = jnp.sum(jnp.where(li == lane, rev, 0))
    bin_index = vstar * L + (L - 1 - lane)
    above = prev + c_at - h_at
    return bin_index, rank - above


def _do_row(row_v, zbuf, cand, h1, h2, h3, o_row):
    ones = jnp.ones((L,), jnp.int32)
    li = lax.iota(jnp.int32, L)

    _zero_hist(h1, H1)
    _zero_hist(h2, H2)
    _zero_hist(h3, H3)

    @plsc.parallel_loop(0, C, L, unroll=8)
    def _(i):
        r = jnp.maximum(row_v[pl.ds(i, L)], 0.0)
        k = plsc.bitcast(r, jnp.int32)
        plsc.addupdate_scatter(
            h1, [jnp.right_shift(k, H2_BITS + H3_BITS)], ones)

    b1, rank2 = _scan_hist(h1, H1, jnp.int32(K))

    def body_b(v, base):
        i = v * L
        r = jnp.maximum(row_v[pl.ds(i, L)], 0.0)
        k = plsc.bitcast(r, jnp.int32)
        hi = jnp.right_shift(k, H2_BITS + H3_BITS)
        meq = hi == b1
        plsc.addupdate_scatter(
            h2, [jnp.bitwise_and(jnp.right_shift(k, H3_BITS), H2 - 1)],
            ones, mask=meq)
        m = hi >= b1
        mi = m.astype(jnp.int32)
        off = plsc.cumsum(mi) - mi
        plsc.store_scatter(cand, [base + off], i + li, mask=m)
        return base + jnp.sum(mi)

    n_cand = lax.fori_loop(0, C // L, body_b, jnp.int32(0), unroll=8)

    b2, rank3 = _scan_hist(h2, H2, rank2)
    prefix = jnp.bitwise_or(lax.shift_left(b1, H2_BITS), b2)

    ntrips = lax.shift_right_logical(n_cand + (L - 1), 4)

    def body_c(t, _):
        i = t * L
        valid = (i + li) < n_cand
        idx = cand[pl.ds(i, L)]
        r = plsc.load_gather(row_v, [idx], mask=valid)
        k = plsc.bitcast(jnp.maximum(r, 0.0), jnp.int32)
        m = jnp.logical_and(valid, jnp.right_shift(k, H3_BITS) == prefix)
        plsc.addupdate_scatter(h3, [jnp.bitwise_and(k, H3 - 1)], ones, mask=m)
        return _

    lax.fori_loop(0, ntrips, body_c, jnp.int32(0))

    b3, rank_eq = _scan_hist(h3, H3, rank3)
    tkey = jnp.bitwise_or(lax.shift_left(prefix, H3_BITS), b3)

    def body_o(t, eqcount):
        i = t * L
        valid = (i + li) < n_cand
        idx = cand[pl.ds(i, L)]
        r = plsc.load_gather(row_v, [idx], mask=valid)
        r = jnp.maximum(r, 0.0)
        k = plsc.bitcast(r, jnp.int32)
        meq = jnp.logical_and(valid, k == tkey)
        meqi = meq.astype(jnp.int32)
        pc = plsc.cumsum(meqi)
        occ = eqcount + pc - meqi
        keep = jnp.logical_and(
            valid,
            jnp.logical_or(k > tkey, jnp.logical_and(meq, occ < rank_eq)))
        plsc.store_scatter(zbuf, [idx], r, mask=keep)
        return eqcount + jnp.sum(meqi)

    lax.fori_loop(0, ntrips, body_o, jnp.int32(0))

    pltpu.sync_copy(zbuf, o_row)

    zeros_f = jnp.zeros((L,), jnp.float32)

    def body_z(t, _):
        i = t * L
        valid = (i + li) < n_cand
        idx = cand[pl.ds(i, L)]
        plsc.store_scatter(zbuf, [idx], zeros_f, mask=valid)
        return _

    lax.fori_loop(0, ntrips, body_z, jnp.int32(0))


def kernel(x):
    mesh = plsc.VectorSubcoreMesh(core_axis_name="c", subcore_axis_name="s")
    cp = pltpu.CompilerParams()
    if "needs_layout_passes" in pltpu.CompilerParams.__dataclass_fields__:
        cp = dataclasses.replace(cp, needs_layout_passes=False)

    @functools.partial(
        pl.kernel,
        out_type=jax.ShapeDtypeStruct((R, C), jnp.float32),
        mesh=mesh,
        compiler_params=cp,
        scratch_types=[
            pltpu.VMEM((C,), jnp.float32),
            pltpu.VMEM((C,), jnp.float32),
            pltpu.VMEM((C,), jnp.int32),
            pltpu.VMEM((H1,), jnp.int32),
            pltpu.VMEM((H2,), jnp.int32),
            pltpu.VMEM((H3,), jnp.int32),
        ],
    )
    def k(x_hbm, o_hbm, row_v, zbuf, cand, h1, h2, h3):
        wid = lax.axis_index("s") * 2 + lax.axis_index("c")

        zeros_f = jnp.zeros((L,), jnp.float32)

        @plsc.parallel_loop(0, C, L, unroll=8)
        def _(i):
            zbuf[pl.ds(i, L)] = zeros_f

        @pl.loop(0, ROWS_PER)
        def _(j):
            row = wid * ROWS_PER + j
            pltpu.sync_copy(x_hbm.at[row], row_v)
            _do_row(row_v, zbuf, cand, h1, h2, h3, o_hbm.at[row])

    return k(x)
